# baseline (speedup 1.0000x reference)
import functools

import jax
import jax.numpy as jnp
from jax import lax
from jax.experimental import pallas as pl
from jax.experimental.pallas import tpu as pltpu, tpu_sc as plsc

N = 10000
E = 320000
D = 128
H = 128
EPS = 1e-5

NW = 32
CHUNK = 128
NCHUNK = 80
NC0 = 136
NC1 = 24
EPAD = NW * NCHUNK * CHUNK
TRASH = N
SLABG = 8
NPAD = 10240
RPT = NPAD // 16

_MESH = plsc.VectorSubcoreMesh(core_axis_name="c", subcore_axis_name="s")


@functools.partial(
    pl.kernel,
    out_type=jax.ShapeDtypeStruct((2 * NPAD,), jnp.float32),
    mesh=_MESH,
    scratch_types=[
        pltpu.VMEM((NCHUNK, CHUNK), jnp.int32),
        pltpu.VMEM((CHUNK,), jnp.float32),
        pltpu.VMEM_SHARED((NPAD,), jnp.float32),
    ],
)
def _sc_deg(dst_hbm, ones_hbm, zeros_hbm, out_hbm, dst_v, ones_v, deg_sh):
    cid = lax.axis_index("c")
    sid = lax.axis_index("s")
    wid = cid * 16 + sid
    pltpu.sync_copy(dst_hbm.at[pl.ds(wid * NCHUNK, NCHUNK)], dst_v)
    pltpu.sync_copy(ones_hbm, ones_v)
    pltpu.sync_copy(zeros_hbm, deg_sh.at[pl.ds(sid * RPT, RPT)])
    plsc.subcore_barrier()

    def body(j, carry):
        pltpu.sync_copy(ones_v, deg_sh.at[dst_v.at[j]], add=True)
        return carry

    lax.fori_loop(0, NCHUNK, body, 0)
    plsc.subcore_barrier()
    pltpu.sync_copy(deg_sh.at[pl.ds(sid * RPT, RPT)],
                    out_hbm.at[pl.ds(cid * NPAD + sid * RPT, RPT)])


@functools.partial(
    pl.kernel,
    out_type=jax.ShapeDtypeStruct((2, NPAD, H), jnp.float32),
    mesh=_MESH,
    scratch_types=[
        pltpu.VMEM((SLABG, CHUNK), jnp.int32),
        pltpu.VMEM((SLABG, CHUNK), jnp.int32),
        pltpu.VMEM((CHUNK, H), jnp.float32),
        pltpu.VMEM((CHUNK, H), jnp.float32),
        pltpu.VMEM_SHARED((NPAD, H), jnp.float32),
        pltpu.SemaphoreType.DMA,
        pltpu.SemaphoreType.DMA,
    ],
)
def _sc_agg(hw_hbm, src_hbm, dst_hbm, zeros_hbm, out_hbm,
            src_v, dst_v, r0, r1, agg_sh, gsem, ssem):
    cid = lax.axis_index("c")
    sid = lax.axis_index("s")
    nc = jnp.where(cid == 0, NC0, NC1)
    gbase = jnp.where(cid == 0, sid * NC0, 16 * NC0 + sid * NC1)
    pltpu.sync_copy(zeros_hbm, agg_sh.at[pl.ds(sid * RPT, RPT)])
    plsc.subcore_barrier()

    rows = (r0, r1)

    def stage(st, carry):
        pltpu.sync_copy(src_hbm.at[pl.ds(gbase + st * SLABG, SLABG)], src_v)
        pltpu.sync_copy(dst_hbm.at[pl.ds(gbase + st * SLABG, SLABG)], dst_v)
        pltpu.async_copy(hw_hbm.at[src_v.at[0]], r0, gsem)

        def pair(p, c2):
            for b in range(2):
                k = p * 2 + b
                pltpu.make_async_copy(hw_hbm.at[src_v.at[k]], rows[b],
                                      gsem).wait()
                pltpu.async_copy(rows[b], agg_sh.at[dst_v.at[k]], ssem,
                                 add=True)

                @pl.when(k >= 1)
                def _():
                    pltpu.make_async_copy(rows[1 - b],
                                          agg_sh.at[dst_v.at[k]], ssem).wait()

                @pl.when(k + 1 < SLABG)
                def _():
                    pltpu.async_copy(hw_hbm.at[src_v.at[k + 1]], rows[1 - b],
                                     gsem)
            return c2

        lax.fori_loop(0, SLABG // 2, pair, 0)
        pltpu.make_async_copy(r0, agg_sh.at[dst_v.at[0]], ssem).wait()
        return carry

    lax.fori_loop(0, nc // SLABG, stage, 0)
    plsc.subcore_barrier()
    pltpu.sync_copy(agg_sh.at[pl.ds(sid * RPT, RPT)],
                    out_hbm.at[cid, pl.ds(sid * RPT, RPT)])


def _tc0_body(x_ref, w_ref, hw_ref):
    hw_ref[...] = jnp.dot(x_ref[...], w_ref[...],
                          preferred_element_type=jnp.float32,
                          precision=jax.lax.Precision.HIGHEST)


def _tc1_body(hw_ref, degp_ref, hwp_ref, dis_ref):
    deg = degp_ref[pl.ds(0, N)] + degp_ref[pl.ds(NPAD, N)]
    dis = jax.lax.rsqrt(deg + 1.0).reshape(N, 1)
    dis_b = jnp.broadcast_to(dis, (N, H))
    hwp_ref[...] = hw_ref[...] * dis_b
    dis_ref[...] = dis_b


def _bn_relu(pre, g, be):
    m = jnp.mean(pre, axis=0, keepdims=True)
    c = pre - m
    v = jnp.mean(c * c, axis=0, keepdims=True)
    return jnp.maximum(c * jax.lax.rsqrt(v + EPS) * g + be, 0.0)


def _tc2_body(aggp_ref, hwp_ref, dis_ref, b_ref, g_ref, be_ref, w2_ref,
              out_ref):
    agg = aggp_ref[0, 0:N, :] + aggp_ref[1, 0:N, :] + hwp_ref[...]
    pre = agg * dis_ref[...] + b_ref[...]
    h = _bn_relu(pre, g_ref[...], be_ref[...])
    out_ref[...] = jnp.dot(
        h, w2_ref[...], preferred_element_type=jnp.float32, precision=jax.lax.Precision.HIGHEST) * dis_ref[...]


def _tc3_body(aggp_ref, hwp_ref, dis_ref, b_ref, g_ref, be_ref, wl_ref,
              bl_ref, out_ref):
    agg = aggp_ref[0, 0:N, :] + aggp_ref[1, 0:N, :] + hwp_ref[...]
    pre = agg * dis_ref[...] + b_ref[...]
    h = _bn_relu(pre, g_ref[...], be_ref[...])
    z = jnp.dot(h, wl_ref[...], preferred_element_type=jnp.float32, precision=jax.lax.Precision.HIGHEST)
    z = z + bl_ref[...]
    out_ref[...] = jax.nn.sigmoid(jnp.maximum(z, 0.0))


@jax.jit
def kernel(x, edge_index, W1, b1, g1, be1, W2, b2, g2, be2, Wl, bl):
    src = edge_index[0]
    dst = edge_index[1]
    pad = EPAD - E
    src_p = jnp.concatenate([src, jnp.zeros((pad,), jnp.int32)])
    dst_p = jnp.concatenate([dst, jnp.full((pad,), TRASH, jnp.int32)])
    src_slab = src_p.reshape(NW * NCHUNK, CHUNK)
    dst_slab = dst_p.reshape(NW * NCHUNK, CHUNK)

    ones1 = jnp.ones((CHUNK,), jnp.float32)
    zeros1 = jnp.zeros((RPT,), jnp.float32)
    zerosH = jnp.zeros((RPT, H), jnp.float32)

    degp = _sc_deg(dst_slab, ones1, zeros1)

    hw1 = pl.pallas_call(
        _tc0_body,
        out_shape=jax.ShapeDtypeStruct((N, H), jnp.float32),
    )(x, W1)

    hw1p, dis_b = pl.pallas_call(
        _tc1_body,
        out_shape=(jax.ShapeDtypeStruct((N, H), jnp.float32),
                   jax.ShapeDtypeStruct((N, H), jnp.float32)),
    )(hw1, degp)

    agg1 = _sc_agg(hw1p, src_slab, dst_slab, zerosH)

    hw2p = pl.pallas_call(
        _tc2_body,
        out_shape=jax.ShapeDtypeStruct((N, H), jnp.float32),
    )(agg1, hw1p, dis_b, b1.reshape(1, H), g1.reshape(1, H),
      be1.reshape(1, H), W2)

    agg2 = _sc_agg(hw2p, src_slab, dst_slab, zerosH)

    out = pl.pallas_call(
        _tc3_body,
        out_shape=jax.ShapeDtypeStruct((N, 1), jnp.float32),
    )(agg2, hw2p, dis_b, b2.reshape(1, H), g2.reshape(1, H),
      be2.reshape(1, H), Wl, bl.reshape(1, 1))

    return out


# speedup vs baseline: 1.0501x; 1.0501x over previous
import functools

import jax
import jax.numpy as jnp
from jax import lax
from jax.experimental import pallas as pl
from jax.experimental.pallas import tpu as pltpu, tpu_sc as plsc

N = 10000
E = 320000
D = 128
H = 128
EPS = 1e-5

NW = 32
CHUNK = 128
NCHUNK = 80
NC0 = 144
NC1 = 16
EPAD = NW * NCHUNK * CHUNK
TRASH = N
SLABG = 16
NPAD = 10240
RPT = NPAD // 16

_MESH = plsc.VectorSubcoreMesh(core_axis_name="c", subcore_axis_name="s")


@functools.partial(
    pl.kernel,
    out_type=jax.ShapeDtypeStruct((2 * NPAD,), jnp.float32),
    mesh=_MESH,
    scratch_types=[
        pltpu.VMEM((NCHUNK, CHUNK), jnp.int32),
        pltpu.VMEM((CHUNK,), jnp.float32),
        pltpu.VMEM_SHARED((NPAD,), jnp.float32),
    ],
)
def _sc_deg(dst_hbm, ones_hbm, zeros_hbm, out_hbm, dst_v, ones_v, deg_sh):
    cid = lax.axis_index("c")
    sid = lax.axis_index("s")
    wid = cid * 16 + sid
    pltpu.sync_copy(dst_hbm.at[pl.ds(wid * NCHUNK, NCHUNK)], dst_v)
    pltpu.sync_copy(ones_hbm, ones_v)
    pltpu.sync_copy(zeros_hbm, deg_sh.at[pl.ds(sid * RPT, RPT)])
    plsc.subcore_barrier()

    def body(j, carry):
        pltpu.sync_copy(ones_v, deg_sh.at[dst_v.at[j]], add=True)
        return carry

    lax.fori_loop(0, NCHUNK, body, 0)
    plsc.subcore_barrier()
    pltpu.sync_copy(deg_sh.at[pl.ds(sid * RPT, RPT)],
                    out_hbm.at[pl.ds(cid * NPAD + sid * RPT, RPT)])


@functools.partial(
    pl.kernel,
    out_type=jax.ShapeDtypeStruct((2, NPAD, H), jnp.float32),
    mesh=_MESH,
    scratch_types=[
        pltpu.VMEM((SLABG, CHUNK), jnp.int32),
        pltpu.VMEM((SLABG, CHUNK), jnp.int32),
        pltpu.VMEM((CHUNK, H), jnp.float32),
        pltpu.VMEM((CHUNK, H), jnp.float32),
        pltpu.VMEM_SHARED((NPAD, H), jnp.float32),
        pltpu.SemaphoreType.DMA,
        pltpu.SemaphoreType.DMA,
    ],
)
def _sc_agg(hw_hbm, src_hbm, dst_hbm, zeros_hbm, out_hbm,
            src_v, dst_v, r0, r1, agg_sh, gsem, ssem):
    cid = lax.axis_index("c")
    sid = lax.axis_index("s")
    nc = jnp.where(cid == 0, NC0, NC1)
    gbase = jnp.where(cid == 0, sid * NC0, 16 * NC0 + sid * NC1)
    pltpu.sync_copy(zeros_hbm, agg_sh.at[pl.ds(sid * RPT, RPT)])
    plsc.subcore_barrier()

    rows = (r0, r1)

    def stage(st, carry):
        pltpu.sync_copy(src_hbm.at[pl.ds(gbase + st * SLABG, SLABG)], src_v)
        pltpu.sync_copy(dst_hbm.at[pl.ds(gbase + st * SLABG, SLABG)], dst_v)
        pltpu.async_copy(hw_hbm.at[src_v.at[0]], r0, gsem)

        def pair(p, c2):
            for b in range(2):
                k = p * 2 + b
                pltpu.make_async_copy(hw_hbm.at[src_v.at[k]], rows[b],
                                      gsem).wait()
                pltpu.async_copy(rows[b], agg_sh.at[dst_v.at[k]], ssem,
                                 add=True)

                @pl.when(k >= 1)
                def _():
                    pltpu.make_async_copy(rows[1 - b],
                                          agg_sh.at[dst_v.at[k]], ssem).wait()

                @pl.when(k + 1 < SLABG)
                def _():
                    pltpu.async_copy(hw_hbm.at[src_v.at[k + 1]], rows[1 - b],
                                     gsem)
            return c2

        lax.fori_loop(0, SLABG // 2, pair, 0)
        pltpu.make_async_copy(r0, agg_sh.at[dst_v.at[0]], ssem).wait()
        return carry

    lax.fori_loop(0, nc // SLABG, stage, 0)
    plsc.subcore_barrier()
    pltpu.sync_copy(agg_sh.at[pl.ds(sid * RPT, RPT)],
                    out_hbm.at[cid, pl.ds(sid * RPT, RPT)])


def _tc0_body(x_ref, w_ref, hw_ref):
    hw_ref[...] = jnp.dot(x_ref[...], w_ref[...],
                          preferred_element_type=jnp.float32,
                          precision=jax.lax.Precision.HIGHEST)


def _tc1_body(hw_ref, degp_ref, hwp_ref, dis_ref):
    deg = degp_ref[pl.ds(0, N)] + degp_ref[pl.ds(NPAD, N)]
    dis = jax.lax.rsqrt(deg + 1.0).reshape(N, 1)
    dis_b = jnp.broadcast_to(dis, (N, H))
    hwp_ref[...] = hw_ref[...] * dis_b
    dis_ref[...] = dis_b


def _bn_relu(pre, g, be):
    m = jnp.mean(pre, axis=0, keepdims=True)
    c = pre - m
    v = jnp.mean(c * c, axis=0, keepdims=True)
    return jnp.maximum(c * jax.lax.rsqrt(v + EPS) * g + be, 0.0)


def _tc2_body(aggp_ref, hwp_ref, dis_ref, b_ref, g_ref, be_ref, w2_ref,
              out_ref):
    agg = aggp_ref[0, 0:N, :] + aggp_ref[1, 0:N, :] + hwp_ref[...]
    pre = agg * dis_ref[...] + b_ref[...]
    h = _bn_relu(pre, g_ref[...], be_ref[...])
    out_ref[...] = jnp.dot(
        h, w2_ref[...], preferred_element_type=jnp.float32, precision=jax.lax.Precision.HIGHEST) * dis_ref[...]


def _tc3_body(aggp_ref, hwp_ref, dis_ref, b_ref, g_ref, be_ref, wl_ref,
              bl_ref, out_ref):
    agg = aggp_ref[0, 0:N, :] + aggp_ref[1, 0:N, :] + hwp_ref[...]
    pre = agg * dis_ref[...] + b_ref[...]
    h = _bn_relu(pre, g_ref[...], be_ref[...])
    z = jnp.dot(h, wl_ref[...], preferred_element_type=jnp.float32, precision=jax.lax.Precision.HIGHEST)
    z = z + bl_ref[...]
    out_ref[...] = jax.nn.sigmoid(jnp.maximum(z, 0.0))


@jax.jit
def kernel(x, edge_index, W1, b1, g1, be1, W2, b2, g2, be2, Wl, bl):
    src = edge_index[0]
    dst = edge_index[1]
    pad = EPAD - E
    src_p = jnp.concatenate([src, jnp.zeros((pad,), jnp.int32)])
    dst_p = jnp.concatenate([dst, jnp.full((pad,), TRASH, jnp.int32)])
    src_slab = src_p.reshape(NW * NCHUNK, CHUNK)
    dst_slab = dst_p.reshape(NW * NCHUNK, CHUNK)

    ones1 = jnp.ones((CHUNK,), jnp.float32)
    zeros1 = jnp.zeros((RPT,), jnp.float32)
    zerosH = jnp.zeros((RPT, H), jnp.float32)

    degp = _sc_deg(dst_slab, ones1, zeros1)

    hw1 = pl.pallas_call(
        _tc0_body,
        out_shape=jax.ShapeDtypeStruct((N, H), jnp.float32),
    )(x, W1)

    hw1p, dis_b = pl.pallas_call(
        _tc1_body,
        out_shape=(jax.ShapeDtypeStruct((N, H), jnp.float32),
                   jax.ShapeDtypeStruct((N, H), jnp.float32)),
    )(hw1, degp)

    agg1 = _sc_agg(hw1p, src_slab, dst_slab, zerosH)

    hw2p = pl.pallas_call(
        _tc2_body,
        out_shape=jax.ShapeDtypeStruct((N, H), jnp.float32),
    )(agg1, hw1p, dis_b, b1.reshape(1, H), g1.reshape(1, H),
      be1.reshape(1, H), W2)

    agg2 = _sc_agg(hw2p, src_slab, dst_slab, zerosH)

    out = pl.pallas_call(
        _tc3_body,
        out_shape=jax.ShapeDtypeStruct((N, 1), jnp.float32),
    )(agg2, hw2p, dis_b, b2.reshape(1, H), g2.reshape(1, H),
      be2.reshape(1, H), Wl, bl.reshape(1, 1))

    return out
